# Initial kernel scaffold; baseline (speedup 1.0000x reference)
import jax
import jax.numpy as jnp
from jax.experimental import pallas as pl


def kernel(agent_triplets, agent_triplet_mask, track_triplets, track_triplet_mask, edges, params):
    raise NotImplementedError("write your pallas kernel here")


# speedup vs baseline: 1.0001x; 1.0001x over previous
import jax
import jax.numpy as jnp
from jax.experimental import pallas as pl

H = 128
HEADS = 4
EDGE_TYPES = [('agent','track'),('track','agent'),('agent','track'),('track','agent'),('agent','agent'),('agent','agent')]


def _layernorm(x, g, b, eps=1e-5):
    m = x.mean(-1, keepdims=True)
    v = x.var(-1, keepdims=True)
    return (x - m) / jnp.sqrt(v + eps) * g + b


def _triplet_encode(p, trip, mask):
    x = jax.nn.relu(trip @ p['W1'] + p['b1']) @ p['W2'] + p['b2']
    x = _layernorm(x, p['ln_g'], p['ln_b'])
    a = jnp.tanh(x @ p['Wa1'] + p['ba1']) @ p['Wa2'] + p['ba2']
    a = jnp.where(mask[..., None], -jnp.inf, a)
    w = jax.nn.softmax(a, axis=1)
    pooled = (x * w).sum(axis=1)
    return pooled @ p['Wo'] + p['bo']


def _gat(p, x_src, x_dst, ei):
    src, dst = ei[0], ei[1]
    n_dst = x_dst.shape[0]
    h_src = (x_src @ p['W']).reshape(-1, HEADS, H)
    h_dst = (x_dst @ p['W']).reshape(-1, HEADS, H)
    a_s = (h_src * p['att_src'][None]).sum(-1)
    a_d = (h_dst * p['att_dst'][None]).sum(-1)
    alpha = jax.nn.leaky_relu(a_s[src] + a_d[dst], negative_slope=0.2)
    m = jax.ops.segment_max(alpha, dst, num_segments=n_dst)
    m = jnp.where(jnp.isfinite(m), m, 0.0)
    ex = jnp.exp(alpha - m[dst])
    den = jax.ops.segment_sum(ex, dst, num_segments=n_dst)
    w = ex / (den[dst] + 1e-16)
    msg = (h_src[src] * w[..., None]).reshape(-1, HEADS * H)
    out = jax.ops.segment_sum(msg, dst, num_segments=n_dst).reshape(n_dst, HEADS, H)
    return out.mean(axis=1) + p['bias']


def _hetero(convs, x, edges):
    outs = {'agent': [], 'track': []}
    for i, (s, d) in enumerate(EDGE_TYPES):
        outs[d].append(_gat(convs[i], x[s], x[d], edges[i]))
    return {k: jnp.mean(jnp.stack(v, 0), axis=0) for k, v in outs.items()}


def _bn(p, x, eps=1e-5):
    return x / jnp.sqrt(1.0 + eps) * p['g'] + p['b']


def _classifier(p, x):
    h = x @ p['Wc1'] + p['bc1']
    h = jax.nn.relu(_layernorm(h, p['ln_g'], p['ln_b']))
    h = jax.nn.relu(h @ p['Wc2'] + p['bc2'])
    return jax.nn.sigmoid(h @ p['Wc3'] + p['bc3'])


def kernel(agent_triplets, agent_triplet_mask, track_triplets, track_triplet_mask, edges, params):
    a = _triplet_encode(params['te_agent'], agent_triplets, agent_triplet_mask)
    t = _triplet_encode(params['te_track'], track_triplets, track_triplet_mask)
    x = {'agent': a, 'track': t}
    x1 = _hetero(params['conv1'], x, edges)
    x1 = {k: _bn(params['norm1'][k], jax.nn.relu(v)) for k, v in x1.items()}
    x2 = _hetero(params['conv2'], x1, edges)
    x2 = {k: _bn(params['norm2'][k], jax.nn.relu(v)) for k, v in x2.items()}
    xf = {k: x2[k] + x1[k] for k in x2}
    return (_classifier(params['cls_agent'], xf['agent']),
            _classifier(params['cls_track'], xf['track']))


# speedup vs baseline: 1.3381x; 1.3380x over previous
import functools
import jax
import jax.numpy as jnp
from jax import lax
from jax.experimental import pallas as pl
from jax.experimental.pallas import tpu as pltpu

H = 128
HEADS = 4
LTRIP = 16
_ETYPES = [('agent', 'track'), ('track', 'agent'), ('agent', 'track'),
           ('track', 'agent'), ('agent', 'agent'), ('agent', 'agent')]
_GROUPS = {'track': (0, 2), 'agent': (1, 3, 4, 5)}

_BT = 1000


def _full(x):
    return pl.BlockSpec(x.shape, lambda i, *_: (0,) * x.ndim)


def _te_body(trip_ref, mask_ref, W1, b1, W2, b2, g, bln, Wa1, ba1, Wa2, ba2,
             Wo, bo, out_ref):
    t2 = trip_ref[...]
    xs = []
    logits = []
    for l in range(LTRIP):
        s = t2[:, l * 8:(l + 1) * 8]
        x = jnp.maximum(s @ W1[...] + b1[...], 0.0) @ W2[...] + b2[...]
        m = x.mean(-1, keepdims=True)
        v = jnp.mean((x - m) ** 2, -1, keepdims=True)
        x = (x - m) / jnp.sqrt(v + 1e-5) * g[...] + bln[...]
        xs.append(x)
        a = jnp.tanh(x @ Wa1[...] + ba1[...])
        a = (a * Wa2[...]).sum(-1, keepdims=True) + ba2[...]
        logits.append(a)
    a = jnp.concatenate(logits, axis=-1)
    a = jnp.where(mask_ref[...] > 0, -jnp.inf, a)
    a = a - a.max(-1, keepdims=True)
    e = jnp.exp(a)
    w = e / e.sum(-1, keepdims=True)
    pooled = xs[0] * w[:, 0:1]
    for l in range(1, LTRIP):
        pooled = pooled + xs[l] * w[:, l:l + 1]
    out_ref[...] = pooled @ Wo[...] + bo[...]


def _triplet_encode(p, trip, mask):
    n = trip.shape[0]
    tf = trip.reshape(n, LTRIP * 8)
    mf = mask.astype(jnp.float32)
    args = (tf, mf,
            p['W1'], p['b1'].reshape(1, H), p['W2'], p['b2'].reshape(1, H),
            p['ln_g'].reshape(1, H), p['ln_b'].reshape(1, H),
            p['Wa1'], p['ba1'].reshape(1, H // 2),
            p['Wa2'].reshape(1, H // 2), p['ba2'].reshape(1, 1),
            p['Wo'], p['bo'].reshape(1, H))
    in_specs = [pl.BlockSpec((_BT, LTRIP * 8), lambda i: (i, 0)),
                pl.BlockSpec((_BT, LTRIP), lambda i: (i, 0))]
    in_specs += [_full(a) for a in args[2:]]
    return pl.pallas_call(
        _te_body,
        grid=(n // _BT,),
        in_specs=in_specs,
        out_specs=pl.BlockSpec((_BT, H), lambda i: (i, 0)),
        out_shape=jax.ShapeDtypeStruct((n, H), jnp.float32),
    )(*args)


def _mm_body(x_ref, w_ref, out_ref):
    out_ref[...] = x_ref[...] @ w_ref[...]


def _matmul(x, w):
    n = x.shape[0]
    return pl.pallas_call(
        _mm_body,
        grid=(n // _BT,),
        in_specs=[pl.BlockSpec((_BT, x.shape[1]), lambda i: (i, 0)), _full(w)],
        out_specs=pl.BlockSpec((_BT, w.shape[1]), lambda i: (i, 0)),
        out_shape=jax.ShapeDtypeStruct((n, w.shape[1]), jnp.float32),
    )(x, w)


def _proj_body(x_ref, w_ref, out_ref):
    out_ref[...] = x_ref[...] @ w_ref[0]


def _proj_stacked(xcat, wstk, sel_fn):
    R, _, C = wstk.shape
    n = 25000
    nb = n // _BT
    return pl.pallas_call(
        _proj_body,
        grid=(nb, R),
        in_specs=[
            pl.BlockSpec((_BT, H), lambda i, r: (sel_fn(r) * nb + i, 0)),
            pl.BlockSpec((1, H, C), lambda i, r: (r, 0, 0)),
        ],
        out_specs=pl.BlockSpec((_BT, C), lambda i, r: (r * nb + i, 0)),
        out_shape=jax.ShapeDtypeStruct((R * n, C), jnp.float32),
    )(xcat, wstk)


def _act1_body(v_ref, cb, s, t, out_ref):
    v = v_ref[...] + cb[...]
    out_ref[...] = jnp.maximum(v, 0.0) * s[...] + t[...]


def _act2_body(v_ref, x1_ref, cb, s, t, out_ref):
    v = v_ref[...] + cb[...]
    out_ref[...] = x1_ref[...] + jnp.maximum(v, 0.0) * s[...] + t[...]


def _act(v, cb, s, t, x1=None):
    n = v.shape[0]
    args = (v,) + (() if x1 is None else (x1,)) + (cb, s, t)
    body = _act1_body if x1 is None else _act2_body
    nvec = 1 if x1 is None else 2
    in_specs = [pl.BlockSpec((_BT, H), lambda i: (i, 0))] * nvec
    in_specs += [_full(a) for a in args[nvec:]]
    return pl.pallas_call(
        body,
        grid=(n // _BT,),
        in_specs=in_specs,
        out_specs=pl.BlockSpec((_BT, H), lambda i: (i, 0)),
        out_shape=jax.ShapeDtypeStruct((n, H), jnp.float32),
    )(*args)


def _cls_body(x_ref, Wc1, bc1, g, b, Wc2, bc2, Wc3, bc3, out_ref):
    h = x_ref[...] @ Wc1[...] + bc1[...]
    m = h.mean(-1, keepdims=True)
    v = jnp.mean((h - m) ** 2, -1, keepdims=True)
    h = jnp.maximum((h - m) / jnp.sqrt(v + 1e-5) * g[...] + b[...], 0.0)
    h = jnp.maximum(h @ Wc2[...] + bc2[...], 0.0)
    o = (h * Wc3[...]).sum(-1, keepdims=True) + bc3[...]
    out_ref[...] = jax.nn.sigmoid(o)


def _classifier(p, x):
    n = x.shape[0]
    args = (x, p['Wc1'], p['bc1'].reshape(1, 64),
            p['ln_g'].reshape(1, 64), p['ln_b'].reshape(1, 64),
            p['Wc2'], p['bc2'].reshape(1, 32),
            p['Wc3'].reshape(1, 32), p['bc3'].reshape(1, 1))
    in_specs = [pl.BlockSpec((_BT, H), lambda i: (i, 0))]
    in_specs += [_full(a) for a in args[1:]]
    return pl.pallas_call(
        _cls_body,
        grid=(n // _BT,),
        in_specs=in_specs,
        out_specs=pl.BlockSpec((_BT, 1), lambda i: (i, 0)),
        out_shape=jax.ShapeDtypeStruct((n, 1), jnp.float32),
    )(*args)


def _half_w(W, k):
    return jnp.concatenate(
        [W[:, h * H + k * 64: h * H + k * 64 + 64] for h in range(HEADS)], axis=1)


def _att_mats(conv):
    ws, wd = [], []
    for p in conv:
        Wr = p['W'].reshape(H, HEADS, H)
        ws.append(jnp.einsum('chk,hk->ch', Wr, p['att_src']))
        wd.append(jnp.einsum('chk,hk->ch', Wr, p['att_dst']))
    return ws, wd


def _prep_edges(edges):
    n = 25000
    E = edges[0].shape[1]
    out = {}
    for gname, rels in _GROUPS.items():
        R = len(rels)
        ept = -(-R * E // (32 * 128)) * 128
        EP = 32 * ept
        pad = EP - R * E
        nden = _den_rows(R)
        srcs = jnp.concatenate(
            [edges[r][0].astype(jnp.int32) + i * n for i, r in enumerate(rels)]
            + [jnp.zeros((pad,), jnp.int32)])
        dstd = jnp.concatenate(
            [edges[r][1].astype(jnp.int32) + i * n for i, r in enumerate(rels)]
            + [jnp.full((pad,), R * n, jnp.int32)])
        dsto = jnp.concatenate(
            [edges[r][1].astype(jnp.int32) for r in rels]
            + [jnp.full((pad,), n, jnp.int32)])
        out[gname] = (srcs, dstd, dsto, R, EP, nden)
    return out


def _den_rows(R):
    n = R * 25000 + 8
    return -(-n // 16) * 16


_NOUT = 25024


def _edge_group(htabs, astab, adtab, eidx):
    srcs, dstd, dsto, R, EP, nden = eidx
    a = astab[srcs] + adtab[dstd]
    al = jnp.maximum(a, 0.2 * a)
    ex = jnp.exp(al)
    den = jax.ops.segment_sum(ex, dstd, num_segments=nden)
    w = ex / (den[dstd] + 1e-16) * (1.0 / (4.0 * R))
    halves = []
    for t in htabs:
        rows = t[srcs]
        msg = sum(w[:, h:h + 1] * rows[:, h * 64:(h + 1) * 64]
                  for h in range(HEADS))
        o = jax.ops.segment_sum(msg, dsto, num_segments=_NOUT)
        halves.append(o[:25000])
    return jnp.concatenate(halves, axis=-1)


def _layer(conv, x_agent, x_track, eidx):
    ws, wd = _att_mats(conv)
    n = 25000
    att_agent = _matmul(x_agent, jnp.concatenate(
        [ws[0], ws[2], ws[4], ws[5], wd[1], wd[3], wd[4], wd[5]], axis=1))
    att_track = _matmul(x_track, jnp.concatenate(
        [ws[1], ws[3], wd[0], wd[2]], axis=1))

    res = {}
    xcat = jnp.concatenate([x_track, x_agent], axis=0)
    for gname, rels in _GROUPS.items():
        R = len(rels)
        sel_fn = (lambda r: 1) if gname == 'track' else (lambda r: r // 2)
        htabs = []
        for k in (0, 1):
            wstk = jnp.stack([_half_w(conv[r]['W'], k) for r in rels])
            htabs.append(_proj_stacked(xcat, wstk, sel_fn))
        if gname == 'track':
            astab = jnp.concatenate([att_agent[:, 0:4], att_agent[:, 4:8]])
            ad = jnp.concatenate([att_track[:, 8:12], att_track[:, 12:16]])
        else:
            astab = jnp.concatenate([att_track[:, 0:4], att_track[:, 4:8],
                                     att_agent[:, 8:12], att_agent[:, 12:16]])
            ad = jnp.concatenate([att_agent[:, 16:20], att_agent[:, 20:24],
                                  att_agent[:, 24:28], att_agent[:, 28:32]])
        nden = eidx[gname][5]
        adtab = jnp.concatenate(
            [ad, jnp.zeros((nden - R * n, 4), jnp.float32)])
        v = _edge_group(htabs, astab, adtab, eidx[gname])
        cb = sum(conv[r]['bias'] for r in rels) / R
        res[gname] = (v, cb.reshape(1, H))
    return res


def _bn_consts(p):
    s = (p['g'] / jnp.sqrt(1.0 + 1e-5)).reshape(1, H)
    t = p['b'].reshape(1, H)
    return s, t


def kernel(agent_triplets, agent_triplet_mask, track_triplets,
           track_triplet_mask, edges, params):
    x_agent = _triplet_encode(params['te_agent'], agent_triplets,
                              agent_triplet_mask)
    x_track = _triplet_encode(params['te_track'], track_triplets,
                              track_triplet_mask)
    eidx = _prep_edges(edges)

    r1 = _layer(params['conv1'], x_agent, x_track, eidx)
    s, t = _bn_consts(params['norm1']['agent'])
    x1_agent = _act(r1['agent'][0], r1['agent'][1], s, t)
    s, t = _bn_consts(params['norm1']['track'])
    x1_track = _act(r1['track'][0], r1['track'][1], s, t)

    r2 = _layer(params['conv2'], x1_agent, x1_track, eidx)
    s, t = _bn_consts(params['norm2']['agent'])
    xf_agent = _act(r2['agent'][0], r2['agent'][1], s, t, x1=x1_agent)
    s, t = _bn_consts(params['norm2']['track'])
    xf_track = _act(r2['track'][0], r2['track'][1], s, t, x1=x1_track)

    return (_classifier(params['cls_agent'], xf_agent),
            _classifier(params['cls_track'], xf_track))


# speedup vs baseline: 2.4928x; 1.8629x over previous
import functools
import jax
import jax.numpy as jnp
from jax import lax
from jax.experimental import pallas as pl
from jax.experimental.pallas import tpu as pltpu
from jax.experimental.pallas import tpu_sc as plsc

H = 128
HEADS = 4
LTRIP = 16
_ETYPES = [('agent', 'track'), ('track', 'agent'), ('agent', 'track'),
           ('track', 'agent'), ('agent', 'agent'), ('agent', 'agent')]
_GROUPS = {'track': (0, 2), 'agent': (1, 3, 4, 5)}

_BT = 1000


def _full(x):
    return pl.BlockSpec(x.shape, lambda i, *_: (0,) * x.ndim)


def _te_body(trip_ref, mask_ref, W1, b1, W2, b2, g, bln, Wa1, ba1, Wa2, ba2,
             Wo, bo, out_ref):
    t2 = trip_ref[...]
    xs = []
    logits = []
    for l in range(LTRIP):
        s = t2[:, l * 8:(l + 1) * 8]
        x = jnp.maximum(s @ W1[...] + b1[...], 0.0) @ W2[...] + b2[...]
        m = x.mean(-1, keepdims=True)
        v = jnp.mean((x - m) ** 2, -1, keepdims=True)
        x = (x - m) / jnp.sqrt(v + 1e-5) * g[...] + bln[...]
        xs.append(x)
        a = jnp.tanh(x @ Wa1[...] + ba1[...])
        a = (a * Wa2[...]).sum(-1, keepdims=True) + ba2[...]
        logits.append(a)
    a = jnp.concatenate(logits, axis=-1)
    a = jnp.where(mask_ref[...] > 0, -jnp.inf, a)
    a = a - a.max(-1, keepdims=True)
    e = jnp.exp(a)
    w = e / e.sum(-1, keepdims=True)
    pooled = xs[0] * w[:, 0:1]
    for l in range(1, LTRIP):
        pooled = pooled + xs[l] * w[:, l:l + 1]
    out_ref[...] = pooled @ Wo[...] + bo[...]


def _triplet_encode(p, trip, mask):
    n = trip.shape[0]
    tf = trip.reshape(n, LTRIP * 8)
    mf = mask.astype(jnp.float32)
    args = (tf, mf,
            p['W1'], p['b1'].reshape(1, H), p['W2'], p['b2'].reshape(1, H),
            p['ln_g'].reshape(1, H), p['ln_b'].reshape(1, H),
            p['Wa1'], p['ba1'].reshape(1, H // 2),
            p['Wa2'].reshape(1, H // 2), p['ba2'].reshape(1, 1),
            p['Wo'], p['bo'].reshape(1, H))
    in_specs = [pl.BlockSpec((_BT, LTRIP * 8), lambda i: (i, 0)),
                pl.BlockSpec((_BT, LTRIP), lambda i: (i, 0))]
    in_specs += [_full(a) for a in args[2:]]
    return pl.pallas_call(
        _te_body,
        grid=(n // _BT,),
        in_specs=in_specs,
        out_specs=pl.BlockSpec((_BT, H), lambda i: (i, 0)),
        out_shape=jax.ShapeDtypeStruct((n, H), jnp.float32),
    )(*args)


def _mm_body(x_ref, w_ref, out_ref):
    out_ref[...] = x_ref[...] @ w_ref[...]


def _matmul(x, w):
    n = x.shape[0]
    return pl.pallas_call(
        _mm_body,
        grid=(n // _BT,),
        in_specs=[pl.BlockSpec((_BT, x.shape[1]), lambda i: (i, 0)), _full(w)],
        out_specs=pl.BlockSpec((_BT, w.shape[1]), lambda i: (i, 0)),
        out_shape=jax.ShapeDtypeStruct((n, w.shape[1]), jnp.float32),
    )(x, w)


def _proj_body(x_ref, w_ref, out_ref):
    out_ref[...] = x_ref[...] @ w_ref[0]


def _proj_stacked(xcat, wstk, sel_fn):
    R, _, C = wstk.shape
    n = 25000
    nb = n // _BT
    return pl.pallas_call(
        _proj_body,
        grid=(nb, R),
        in_specs=[
            pl.BlockSpec((_BT, H), lambda i, r: (sel_fn(r) * nb + i, 0)),
            pl.BlockSpec((1, H, C), lambda i, r: (r, 0, 0)),
        ],
        out_specs=pl.BlockSpec((_BT, C), lambda i, r: (r * nb + i, 0)),
        out_shape=jax.ShapeDtypeStruct((R * n, C), jnp.float32),
    )(xcat, wstk)


def _act1_body(v_ref, cb, s, t, out_ref):
    v = v_ref[...] + cb[...]
    out_ref[...] = jnp.maximum(v, 0.0) * s[...] + t[...]


def _act2_body(v_ref, x1_ref, cb, s, t, out_ref):
    v = v_ref[...] + cb[...]
    out_ref[...] = x1_ref[...] + jnp.maximum(v, 0.0) * s[...] + t[...]


def _act(v, cb, s, t, x1=None):
    n = v.shape[0]
    args = (v,) + (() if x1 is None else (x1,)) + (cb, s, t)
    body = _act1_body if x1 is None else _act2_body
    nvec = 1 if x1 is None else 2
    in_specs = [pl.BlockSpec((_BT, H), lambda i: (i, 0))] * nvec
    in_specs += [_full(a) for a in args[nvec:]]
    return pl.pallas_call(
        body,
        grid=(n // _BT,),
        in_specs=in_specs,
        out_specs=pl.BlockSpec((_BT, H), lambda i: (i, 0)),
        out_shape=jax.ShapeDtypeStruct((n, H), jnp.float32),
    )(*args)


def _cls_body(x_ref, Wc1, bc1, g, b, Wc2, bc2, Wc3, bc3, out_ref):
    h = x_ref[...] @ Wc1[...] + bc1[...]
    m = h.mean(-1, keepdims=True)
    v = jnp.mean((h - m) ** 2, -1, keepdims=True)
    h = jnp.maximum((h - m) / jnp.sqrt(v + 1e-5) * g[...] + b[...], 0.0)
    h = jnp.maximum(h @ Wc2[...] + bc2[...], 0.0)
    o = (h * Wc3[...]).sum(-1, keepdims=True) + bc3[...]
    out_ref[...] = jax.nn.sigmoid(o)


def _classifier(p, x):
    n = x.shape[0]
    args = (x, p['Wc1'], p['bc1'].reshape(1, 64),
            p['ln_g'].reshape(1, 64), p['ln_b'].reshape(1, 64),
            p['Wc2'], p['bc2'].reshape(1, 32),
            p['Wc3'].reshape(1, 32), p['bc3'].reshape(1, 1))
    in_specs = [pl.BlockSpec((_BT, H), lambda i: (i, 0))]
    in_specs += [_full(a) for a in args[1:]]
    return pl.pallas_call(
        _cls_body,
        grid=(n // _BT,),
        in_specs=in_specs,
        out_specs=pl.BlockSpec((_BT, 1), lambda i: (i, 0)),
        out_shape=jax.ShapeDtypeStruct((n, 1), jnp.float32),
    )(*args)


def _half_w(W, k):
    return jnp.concatenate(
        [W[:, h * H + k * 32: h * H + k * 32 + 32] for h in range(HEADS)],
        axis=1)


def _att_mats(conv):
    ws, wd = [], []
    for p in conv:
        Wr = p['W'].reshape(H, HEADS, H)
        ws.append(jnp.einsum('chk,hk->ch', Wr, p['att_src']))
        wd.append(jnp.einsum('chk,hk->ch', Wr, p['att_dst']))
    return ws, wd


def _prep_edges(edges):
    n = 25000
    E = edges[0].shape[1]
    out = {}
    for gname, rels in _GROUPS.items():
        R = len(rels)
        ept = -(-R * E // (32 * 2048)) * 2048
        EP = 32 * ept
        pad = EP - R * E
        srcs = jnp.concatenate(
            [edges[r][0].astype(jnp.int32) + i * n for i, r in enumerate(rels)]
            + [jnp.zeros((pad,), jnp.int32)]).reshape(EP // 128, 128)
        dstg = jnp.concatenate(
            [edges[r][1].astype(jnp.int32) + i * n for i, r in enumerate(rels)]
            + [jnp.full((pad,), R * n, jnp.int32)])
        p = jnp.clip(dstg // (2 * n), 0, R // 2 - 1)
        dstl = (dstg - p * 2 * n).reshape(EP // 128, 128)
        dstb = (p * _NDEN_P + dstg - p * 2 * n).reshape(EP // 128, 128)
        dsto = jnp.concatenate(
            [edges[r][1].astype(jnp.int32) for r in rels]
            + [jnp.full((pad,), n, jnp.int32)]).reshape(EP // 128, 128)
        out[gname] = (srcs, dstg.reshape(EP // 128, 128), dstl, dstb, dsto,
                      R, EP)
    return out


_NOUT = 25088


_NC, _NS, _NL = 2, 16, 16


def _viota():
    return lax.iota(jnp.int32, _NL)


_NDEN_P = 50176


@functools.lru_cache(maxsize=None)
def _den_kernel(R, EP):
    npass = R // 2
    zr = _NDEN_P // _NS
    hzr = zr // 2
    ept = EP // (_NC * _NS)
    eptp = ept // npass
    cha = 1024
    nch = eptp // cha
    mesh = plsc.VectorSubcoreMesh(core_axis_name="c", subcore_axis_name="s")

    @functools.partial(
        pl.kernel, mesh=mesh,
        compiler_params=pltpu.CompilerParams(use_tc_tiling_on_sc=False),
        out_type=[jax.ShapeDtypeStruct((EP, 16), jnp.float32),
                  jax.ShapeDtypeStruct((npass * _NDEN_P, 16), jnp.float32),
                  jax.ShapeDtypeStruct((npass * _NDEN_P, 16), jnp.float32)],
        scratch_types=[
            pltpu.VMEM((cha // 128, 128), jnp.int32),
            pltpu.VMEM((cha // 128, 128), jnp.int32),
            pltpu.VMEM((cha // 128, 128), jnp.int32),
            pltpu.VMEM((cha, 16), jnp.float32),
            pltpu.VMEM((cha, 16), jnp.float32),
            pltpu.VMEM((cha, 16), jnp.float32),
            pltpu.VMEM((hzr, 16), jnp.float32),
            pltpu.VMEM_SHARED((_NDEN_P, 16), jnp.float32),
            pltpu.SemaphoreType.DMA,
        ])
    def k(src_hbm, dstg_hbm, dstl_hbm, as_hbm, ad_hbm,
          ex_hbm, den0_hbm, den1_hbm,
          sidx, gidx, lidx, asv, adv, exv, zb, den_sh, sem):
        cid = lax.axis_index("c")
        sid = lax.axis_index("s")
        wid = cid * _NS + sid
        iot = _viota()
        lane_ok = iot < 4
        zv = jnp.zeros((_NL,), jnp.float32)

        def zzb(i, c):
            zb[i, :] = zv
            return c
        lax.fori_loop(0, hzr, zzb, 0)

        for p in range(npass):
            for half in range(2):
                zo = pl.multiple_of(sid * zr + half * hzr, 8)
                pltpu.sync_copy(zb, den_sh.at[pl.ds(zo, hzr)])
            plsc.subcore_barrier()

            def chunk(c, carry):
                base = pl.multiple_of(
                    p * (EP // npass) + wid * eptp + c * cha, cha)
                brow = pl.multiple_of(base // 128, cha // 128)
                pltpu.sync_copy(src_hbm.at[pl.ds(brow, cha // 128)], sidx)
                pltpu.sync_copy(dstg_hbm.at[pl.ds(brow, cha // 128)], gidx)
                pltpu.sync_copy(dstl_hbm.at[pl.ds(brow, cha // 128)], lidx)
                cps = []
                for j in range(cha // 128):
                    cps.append(pltpu.async_copy(
                        as_hbm.at[sidx.at[j]],
                        asv.at[pl.ds(j * 128, 128)], sem))
                    cps.append(pltpu.async_copy(
                        ad_hbm.at[gidx.at[j]],
                        adv.at[pl.ds(j * 128, 128)], sem))
                for cp in cps:
                    cp.wait()

                def cbody(e, cc):
                    t = asv[e, :] + adv[e, :]
                    t = jnp.maximum(t, 0.2 * t)
                    exv[e, :] = jnp.where(lane_ok, jnp.exp(t), 0.0)
                    return cc
                lax.fori_loop(0, cha, cbody, 0)
                pltpu.sync_copy(exv, ex_hbm.at[pl.ds(base, cha)])
                for j in range(cha // 128):
                    pltpu.sync_copy(exv.at[pl.ds(j * 128, 128)],
                                    den_sh.at[lidx.at[j]], add=True)
                return carry
            lax.fori_loop(0, nch, chunk, 0)

            plsc.subcore_barrier()
            for half in range(2):
                zo = pl.multiple_of(sid * zr + half * hzr, 8)
                po = pl.multiple_of(p * _NDEN_P + zo, 8)
                pltpu.sync_copy(den_sh.at[pl.ds(zo, hzr)], zb)

                @pl.when(cid == 0)
                def _():
                    pltpu.sync_copy(zb, den0_hbm.at[pl.ds(po, hzr)])

                @pl.when(cid == 1)
                def _():
                    pltpu.sync_copy(zb, den1_hbm.at[pl.ds(po, hzr)])

    return k


@functools.lru_cache(maxsize=None)
def _msg_kernel(R, EP):
    ept = EP // (_NC * _NS)
    ch = 128
    nmac = ept // (8 * ch)
    zro = _NOUT // _NS
    hzro = zro // 2
    inv = 1.0 / (4.0 * R)
    mesh = plsc.VectorSubcoreMesh(core_axis_name="c", subcore_axis_name="s")

    @functools.partial(
        pl.kernel, mesh=mesh,
        compiler_params=pltpu.CompilerParams(use_tc_tiling_on_sc=False),
        out_type=[jax.ShapeDtypeStruct((_NOUT, 32), jnp.float32),
                  jax.ShapeDtypeStruct((_NOUT, 32), jnp.float32)],
        scratch_types=[
            pltpu.VMEM((8, 128), jnp.int32),
            pltpu.VMEM((8, 128), jnp.int32),
            pltpu.VMEM((8, 128), jnp.int32),
            pltpu.VMEM((ch, 128), jnp.float32),
            pltpu.VMEM((ch, 16), jnp.float32),
            pltpu.VMEM((ch, 16), jnp.float32),
            pltpu.VMEM((ch, 16), jnp.float32),
            pltpu.VMEM((ch, 32), jnp.float32),
            pltpu.VMEM((hzro, 32), jnp.float32),
            pltpu.VMEM_SHARED((_NOUT, 32), jnp.float32),
            pltpu.SemaphoreType.DMA,
        ])
    def k(src_hbm, dstb_hbm, dsto_hbm, ex_hbm, den0_hbm, den1_hbm, htab_hbm,
          out0_hbm, out1_hbm,
          sidx, didx, oidx, rows, exv, d0v, d1v, msgv, zb, out_sh, sem):
        cid = lax.axis_index("c")
        sid = lax.axis_index("s")
        wid = cid * _NS + sid
        zv = jnp.zeros((_NL,), jnp.float32)

        def zzb(i, c):
            for f in range(2):
                zb[i, pl.ds(f * 16, 16)] = zv
            return c
        lax.fori_loop(0, hzro, zzb, 0)
        pltpu.sync_copy(
            zb, out_sh.at[pl.ds(pl.multiple_of(sid * zro, 8), hzro)])
        pltpu.sync_copy(
            zb, out_sh.at[pl.ds(pl.multiple_of(sid * zro + hzro, 8), hzro)])
        plsc.subcore_barrier()

        def macro(m, carry):
            mbase = pl.multiple_of(wid * ept + m * 8 * ch, 8 * ch)
            mrow = pl.multiple_of(mbase // 128, 8)
            pltpu.sync_copy(src_hbm.at[pl.ds(mrow, 8)], sidx)
            pltpu.sync_copy(dstb_hbm.at[pl.ds(mrow, 8)], didx)
            pltpu.sync_copy(dsto_hbm.at[pl.ds(mrow, 8)], oidx)
            for j in range(8):
                base = pl.multiple_of(mbase + j * ch, ch)
                cps = [pltpu.async_copy(htab_hbm.at[sidx.at[j]], rows, sem),
                       pltpu.async_copy(den0_hbm.at[didx.at[j]], d0v, sem),
                       pltpu.async_copy(den1_hbm.at[didx.at[j]], d1v, sem)]
                pltpu.sync_copy(ex_hbm.at[pl.ds(base, ch)], exv)
                for cp in cps:
                    cp.wait()

                def ebody(e, cc):
                    wrow = exv[e, :] / (d0v[e, :] + d1v[e, :] + 1e-16) * inv
                    wsp = [jnp.take(wrow, jnp.full((_NL,), h, jnp.int32))
                           for h in range(4)]
                    for f in range(2):
                        acc = wsp[0] * rows[e, pl.ds(f * 16, 16)]
                        for h in range(1, 4):
                            acc = acc + wsp[h] * rows[e,
                                                      pl.ds(h * 32 + f * 16,
                                                            16)]
                        msgv[e, pl.ds(f * 16, 16)] = acc
                    return cc
                lax.fori_loop(0, ch, ebody, 0)
                pltpu.sync_copy(msgv, out_sh.at[oidx.at[j]], add=True)
            return carry
        lax.fori_loop(0, nmac, macro, 0)

        plsc.subcore_barrier()
        for half in range(2):
            sl = pl.ds(pl.multiple_of(sid * zro + half * hzro, 8), hzro)
            pltpu.sync_copy(out_sh.at[sl], zb)

            @pl.when(cid == 0)
            def _():
                pltpu.sync_copy(zb, out0_hbm.at[sl])

            @pl.when(cid == 1)
            def _():
                pltpu.sync_copy(zb, out1_hbm.at[sl])

    return k


def _edge_group(htabs, astab, adtab, eidx):
    srcs, dstg, dstl, dstb, dsto, R, EP = eidx
    ex, den0, den1 = _den_kernel(R, EP)(srcs, dstg, dstl, astab, adtab)
    quarters = []
    for t in htabs:
        o0, o1 = _msg_kernel(R, EP)(srcs, dstb, dsto, ex, den0, den1, t)
        quarters.append(o0[:25000] + o1[:25000])
    return jnp.concatenate(quarters, axis=-1)


def _layer(conv, x_agent, x_track, eidx):
    ws, wd = _att_mats(conv)
    n = 25000
    att_agent = _matmul(x_agent, jnp.concatenate(
        [ws[0], ws[2], ws[4], ws[5], wd[1], wd[3], wd[4], wd[5]], axis=1))
    att_track = _matmul(x_track, jnp.concatenate(
        [ws[1], ws[3], wd[0], wd[2]], axis=1))

    res = {}
    xcat = jnp.concatenate([x_track, x_agent], axis=0)
    for gname, rels in _GROUPS.items():
        R = len(rels)
        sel_fn = (lambda r: 1) if gname == 'track' else (lambda r: r // 2)
        htabs = []
        for k in range(4):
            wstk = jnp.stack([_half_w(conv[r]['W'], k) for r in rels])
            htabs.append(_proj_stacked(xcat, wstk, sel_fn))
        if gname == 'track':
            astab = jnp.concatenate([att_agent[:, 0:4], att_agent[:, 4:8]])
            ad = jnp.concatenate([att_track[:, 8:12], att_track[:, 12:16]])
        else:
            astab = jnp.concatenate([att_track[:, 0:4], att_track[:, 4:8],
                                     att_agent[:, 8:12], att_agent[:, 12:16]])
            ad = jnp.concatenate([att_agent[:, 16:20], att_agent[:, 20:24],
                                  att_agent[:, 24:28], att_agent[:, 28:32]])
        nad = (R // 2) * _NDEN_P
        astab = jnp.pad(astab, ((0, 0), (0, 12)))
        adtab = jnp.pad(ad, ((0, nad - R * n), (0, 12)))
        v = _edge_group(htabs, astab, adtab, eidx[gname])
        cb = sum(conv[r]['bias'] for r in rels) / R
        res[gname] = (v, cb.reshape(1, H))
    return res


def _bn_consts(p):
    s = (p['g'] / jnp.sqrt(1.0 + 1e-5)).reshape(1, H)
    t = p['b'].reshape(1, H)
    return s, t


def kernel(agent_triplets, agent_triplet_mask, track_triplets,
           track_triplet_mask, edges, params):
    x_agent = _triplet_encode(params['te_agent'], agent_triplets,
                              agent_triplet_mask)
    x_track = _triplet_encode(params['te_track'], track_triplets,
                              track_triplet_mask)
    eidx = _prep_edges(edges)

    r1 = _layer(params['conv1'], x_agent, x_track, eidx)
    s, t = _bn_consts(params['norm1']['agent'])
    x1_agent = _act(r1['agent'][0], r1['agent'][1], s, t)
    s, t = _bn_consts(params['norm1']['track'])
    x1_track = _act(r1['track'][0], r1['track'][1], s, t)

    r2 = _layer(params['conv2'], x1_agent, x1_track, eidx)
    s, t = _bn_consts(params['norm2']['agent'])
    xf_agent = _act(r2['agent'][0], r2['agent'][1], s, t, x1=x1_agent)
    s, t = _bn_consts(params['norm2']['track'])
    xf_track = _act(r2['track'][0], r2['track'][1], s, t, x1=x1_track)

    return (_classifier(params['cls_agent'], xf_agent),
            _classifier(params['cls_track'], xf_track))
